# baseline (speedup 1.0000x reference)
import jax
import jax.numpy as jnp
from jax.experimental import pallas as pl
from jax.experimental.pallas import tpu as pltpu

_BIG = 1e11
_NSUM = 8
_NSPLIT = 2


def _body(func_ref, *refs):
    state_refs = refs[:_NSPLIT]
    out_ref, cnt_ref = refs[_NSPLIT:]
    g = func_ref.shape[0]
    hs = state_refs[0].shape[-1]
    for i in range(g):
        f = func_ref[i]
        s = f.shape[-1]
        lab = jax.lax.broadcasted_iota(jnp.int32, (_NSUM, s), 0)
        fb = jnp.broadcast_to(f, (_NSUM, s))
        eq = jnp.where(fb == lab + 1, 1.0, 0.0)
        nonpad = jnp.where(fb != 0, 1.0, 0.0)
        maskf = jnp.where(lab == 5, nonpad, eq)
        counts = jnp.sum(maskf, axis=1, keepdims=True)
        denom = jnp.where(counts > 0, counts, jnp.float32(_BIG))
        recip = 1.0 / denom
        for j in range(_NSPLIT):
            sums = jax.lax.dot_general(
                maskf, state_refs[j][i],
                dimension_numbers=(((1,), (0,)), ((), ())),
                preferred_element_type=jnp.float32,
            )
            out_ref[i, :, j * hs:(j + 1) * hs] = sums * recip
        cnt_ref[i] = jnp.broadcast_to(counts, (_NSUM, 128))


def _segmean(state, func3, rows_per_step):
    r, s, h = state.shape
    g = rows_per_step
    assert r % g == 0 and h % _NSPLIT == 0
    hs = h // _NSPLIT
    out_shape = (
        jax.ShapeDtypeStruct((r, _NSUM, h), jnp.float32),
        jax.ShapeDtypeStruct((r, _NSUM, 128), jnp.float32),
    )
    state_specs = [
        pl.BlockSpec((g, s, hs), lambda i, j=j: (i, 0, j))
        for j in range(_NSPLIT)
    ]
    return pl.pallas_call(
        _body,
        grid=(r // g,),
        in_specs=[pl.BlockSpec((g, 1, s), lambda i: (i, 0, 0))] + state_specs,
        out_specs=(
            pl.BlockSpec((g, _NSUM, h), lambda i: (i, 0, 0)),
            pl.BlockSpec((g, _NSUM, 128), lambda i: (i, 0, 0)),
        ),
        out_shape=out_shape,
        compiler_params=pltpu.CompilerParams(
            dimension_semantics=("parallel",),
        ),
        name="segmean",
    )(func3, *([state] * _NSPLIT))


def kernel(tarsent_state, tar_func, refsent_state, ref_func):
    b, ts, h = tarsent_state.shape
    _, d, rs, _ = refsent_state.shape

    tar_out, tar_cnt = _segmean(tarsent_state, tar_func.reshape(b, 1, ts), 1)
    ref_out, ref_cnt = _segmean(
        refsent_state.reshape(b * d, rs, h),
        ref_func.reshape(b * d, 1, rs), 8)

    tar_counts = tar_cnt[:, :, 0]
    tar_aug = tar_out[:, :5, :]
    tar_aug_mask = tar_counts[:, :5] > 0
    tarpaper = tar_out[:, 5, :]
    tar_mask2 = tar_counts[:, 5] > 0

    ref_out = ref_out.reshape(b, d, _NSUM, h)
    ref_counts = ref_cnt[:, :, 0].reshape(b, d, _NSUM)
    ref_aug = ref_out[:, :, :5, :]
    ref_aug_mask = ref_counts[:, :, :5] > 0
    refpaper = ref_out[:, :, 5, :]
    ref_mask2 = ref_counts[:, :, 5] > 0

    return (tar_aug, tar_aug_mask, ref_aug, ref_aug_mask,
            tarpaper, tar_mask2, refpaper, ref_mask2)


# speedup vs baseline: 1.0613x; 1.0613x over previous
import jax
import jax.numpy as jnp
from jax.experimental import pallas as pl
from jax.experimental.pallas import tpu as pltpu

_BIG = 1e11
_NSUM = 8


def _body(func_ref, state_ref, out_ref):
    g = func_ref.shape[0]
    for i in range(g):
        f = func_ref[i]
        s = f.shape[-1]
        lab = jax.lax.broadcasted_iota(jnp.int32, (_NSUM, s), 0)
        fb = jnp.broadcast_to(f, (_NSUM, s))
        eq = jnp.where(fb == lab + 1, 1.0, 0.0)
        nonpad = jnp.where(fb != 0, 1.0, 0.0)
        maskf = jnp.where(lab == 5, nonpad, eq)
        counts = jnp.sum(maskf, axis=1, keepdims=True)
        denom = jnp.where(counts > 0, counts, jnp.float32(_BIG))
        sums = jax.lax.dot_general(
            maskf, state_ref[i],
            dimension_numbers=(((1,), (0,)), ((), ())),
            preferred_element_type=jnp.float32,
        )
        out_ref[i] = sums * (1.0 / denom)


def _segmean(state, func3, rows_per_step):
    r, s, h = state.shape
    g = rows_per_step
    assert r % g == 0
    return pl.pallas_call(
        _body,
        grid=(r // g,),
        in_specs=[
            pl.BlockSpec((g, 1, s), lambda i: (i, 0, 0)),
            pl.BlockSpec((g, s, h), lambda i: (i, 0, 0)),
        ],
        out_specs=pl.BlockSpec((g, _NSUM, h), lambda i: (i, 0, 0)),
        out_shape=jax.ShapeDtypeStruct((r, _NSUM, h), jnp.float32),
        compiler_params=pltpu.CompilerParams(
            dimension_semantics=("parallel",),
        ),
        name="segmean",
    )(func3, state)


def _masks(func):
    labels = jnp.arange(1, 6, dtype=func.dtype)
    aug_mask = jnp.any(func[..., :, None] == labels, axis=-2)
    return aug_mask, jnp.any(func != 0, axis=-1)


def kernel(tarsent_state, tar_func, refsent_state, ref_func):
    b, ts, h = tarsent_state.shape
    _, d, rs, _ = refsent_state.shape

    tar_out = _segmean(tarsent_state, tar_func.reshape(b, 1, ts), 2)
    ref_out = _segmean(
        refsent_state.reshape(b * d, rs, h),
        ref_func.reshape(b * d, 1, rs), 8).reshape(b, d, _NSUM, h)

    tar_aug_mask, tar_mask2 = _masks(tar_func)
    ref_aug_mask, ref_mask2 = _masks(ref_func)

    tar_aug = tar_out[:, :5, :]
    tarpaper = tar_out[:, 5, :]
    ref_aug = ref_out[:, :, :5, :]
    refpaper = ref_out[:, :, 5, :]

    return (tar_aug, tar_aug_mask, ref_aug, ref_aug_mask,
            tarpaper, tar_mask2, refpaper, ref_mask2)


# speedup vs baseline: 1.1061x; 1.0422x over previous
import jax
import jax.numpy as jnp
from jax.experimental import pallas as pl
from jax.experimental.pallas import tpu as pltpu

_BIG = 1e11
_NSUM = 8


def _body(func_ref, state_ref, aug_ref, paper_ref):
    g = func_ref.shape[1]
    for i in range(g):
        f = func_ref[0, i]
        s = f.shape[-1]
        lab = jax.lax.broadcasted_iota(jnp.int32, (_NSUM, s), 0)
        fb = jnp.broadcast_to(f, (_NSUM, s))
        eq = jnp.where(fb == lab + 1, 1.0, 0.0)
        nonpad = jnp.where(fb != 0, 1.0, 0.0)
        maskf = jnp.where(lab == 5, nonpad, eq)
        counts = jnp.sum(maskf, axis=1, keepdims=True)
        denom = jnp.where(counts > 0, counts, jnp.float32(_BIG))
        sums = jax.lax.dot_general(
            maskf, state_ref[0, i],
            dimension_numbers=(((1,), (0,)), ((), ())),
            preferred_element_type=jnp.float32,
        )
        scaled = sums * (1.0 / denom)
        aug_ref[0, i] = scaled[:5]
        paper_ref[0, i] = scaled[5:6]


def _segmean(state4, func4, g):
    p, q, s, h = state4.shape
    qg = q // g
    assert q % g == 0

    def idx4(i):
        return (i // qg, i % qg, 0, 0)

    return pl.pallas_call(
        _body,
        grid=(p * qg,),
        in_specs=[
            pl.BlockSpec((1, g, 1, s), idx4),
            pl.BlockSpec((1, g, s, h), idx4),
        ],
        out_specs=(
            pl.BlockSpec((1, g, 5, h), idx4),
            pl.BlockSpec((1, g, 1, h), idx4),
        ),
        out_shape=(
            jax.ShapeDtypeStruct((p, q, 5, h), jnp.float32),
            jax.ShapeDtypeStruct((p, q, 1, h), jnp.float32),
        ),
        compiler_params=pltpu.CompilerParams(
            dimension_semantics=("parallel",),
        ),
        name="segmean",
    )(func4, state4)


def _masks(func):
    labels = jnp.arange(1, 6, dtype=func.dtype)
    aug_mask = jnp.any(func[..., :, None] == labels, axis=-2)
    return aug_mask, jnp.any(func != 0, axis=-1)


def kernel(tarsent_state, tar_func, refsent_state, ref_func):
    b, ts, h = tarsent_state.shape
    _, d, rs, _ = refsent_state.shape

    tar_aug, tarpaper = _segmean(
        tarsent_state.reshape(1, b, ts, h),
        tar_func.reshape(1, b, 1, ts), 2)
    ref_aug, refpaper = _segmean(
        refsent_state, ref_func.reshape(b, d, 1, rs), 8)

    tar_aug_mask, tar_mask2 = _masks(tar_func)
    ref_aug_mask, ref_mask2 = _masks(ref_func)

    return (tar_aug.reshape(b, 5, h), tar_aug_mask, ref_aug, ref_aug_mask,
            tarpaper.reshape(b, h), tar_mask2, refpaper.reshape(b, d, h),
            ref_mask2)
